# baseline (speedup 1.0000x reference)
import functools

import jax
import jax.numpy as jnp
from jax import lax
from jax.experimental import pallas as pl
from jax.experimental.pallas import tpu as pltpu
from jax.experimental.pallas import tpu_sc as plsc

_F32 = jnp.float32
_HI = jax.lax.Precision.HIGHEST


def _bmm(a, b):
    return lax.dot_general(a.astype(jnp.bfloat16), b.astype(jnp.bfloat16),
                           (((1,), (0,)), ((), ())),
                           preferred_element_type=_F32)


def _mm(x, w, prec=None):
    if prec is not None:
        return lax.dot_general(x, w, (((1,), (0,)), ((), ())),
                               preferred_element_type=_F32, precision=prec)
    return _bmm(x, w)


def _b16(x):
    return x.astype(jnp.bfloat16).astype(_F32)


def _relu(x):
    return jnp.maximum(x, 0.0)


def _ln(x):
    m = jnp.mean(x, axis=-1, keepdims=True)
    v = jnp.mean((x - m) * (x - m), axis=-1, keepdims=True)
    return (x - m) / jnp.sqrt(v + 1e-5)


def _run_layers(x, Ws, bs, norm, final_act, prec=None):
    n = len(Ws)
    for i in range(n):
        x = _mm(x, Ws[i], prec=prec)
        x = x + bs[i]
        if i < n - 1 or final_act:
            if norm:
                x = _ln(x)
            x = _relu(x)
    return x


def _knn_body(pts_t_ref, q_ref, idx_ref, *, K):
    p = pts_t_ref[0]
    q = q_ref[0]
    pn = jnp.sum(p * p, axis=0, keepdims=True)
    qn = jnp.sum(q * q, axis=1, keepdims=True)
    qp = _mm(_b16(q), _b16(p), prec=_HI)
    d2 = jnp.maximum(qn + pn - 2.0 * qp, 0.0)
    TQ = q.shape[0]
    N = p.shape[1]
    J = 12
    GW = 128
    G = N // GW
    iota_g = lax.broadcasted_iota(jnp.int32, (TQ, GW), 1)
    groups = [d2[:, g * GW:(g + 1) * GW] for g in range(G)]
    ext_v, ext_i = [], []
    for _ in range(J):
        for g in range(G):
            dg = groups[g]
            vg = jnp.min(dg, axis=1, keepdims=True)
            ag = jnp.argmin(dg, axis=1).astype(jnp.int32)[:, None]
            ext_v.append(vg)
            ext_i.append(ag.astype(_F32) + float(g * GW))
            groups[g] = jnp.where(iota_g == ag, jnp.inf, dg)
    EV = jnp.concatenate(ext_v, axis=1)
    EI = jnp.concatenate(ext_i, axis=1)
    lane2 = lax.broadcasted_iota(jnp.int32, EV.shape, 1).astype(_F32)
    cols = []
    t32 = None
    for k in range(K):
        v = jnp.min(EV, axis=1, keepdims=True)
        pos = jnp.min(jnp.where(EV == v, lane2, float(J * G)),
                      axis=1, keepdims=True)
        sel = lane2 == pos
        cols.append(jnp.sum(jnp.where(sel, EI, 0.0), axis=1, keepdims=True)
                    .astype(jnp.int32))
        if k == K - 1:
            t32 = v
        EV = jnp.where(sel, jnp.inf, EV)
    idx_fast = jnp.concatenate(cols, axis=1)

    drem = jnp.concatenate(groups, axis=1)
    nbad = jnp.sum(jnp.where(drem < t32, 1, 0).astype(jnp.int32))

    @pl.when(nbad == 0)
    def _():
        idx_ref[0] = idx_fast

    @pl.when(nbad > 0)
    def _():
        dd = jnp.maximum(qn + pn - 2.0 * _mm(_b16(q), _b16(p), prec=_HI), 0.0)
        iota = lax.broadcasted_iota(jnp.int32, dd.shape, 1)
        slow = []
        for _k in range(K):
            am = jnp.argmin(dd, axis=1).astype(jnp.int32)[:, None]
            slow.append(am)
            dd = jnp.where(iota == am, jnp.inf, dd)
        idx_ref[0] = jnp.concatenate(slow, axis=1)


def _knn(pts_t, q_pad, K, TQ):
    B, _, N = pts_t.shape
    M = q_pad.shape[1]
    grid = (B, M // TQ)
    return pl.pallas_call(
        functools.partial(_knn_body, K=K),
        grid=grid,
        in_specs=[
            pl.BlockSpec((1, 8, N), lambda b, t: (b, 0, 0)),
            pl.BlockSpec((1, TQ, 8), lambda b, t: (b, t, 0)),
        ],
        out_specs=pl.BlockSpec((1, TQ, K), lambda b, t: (b, t, 0)),
        out_shape=jax.ShapeDtypeStruct((B, M, K), jnp.int32),
    )(pts_t, q_pad)


def _sc_gather(table, idx_flat, chunk):
    V, D = table.shape
    Bidx = idx_flat.shape[0]
    info = plsc.get_sparse_core_info()
    nc = info.num_cores
    nw = nc * info.num_subcores
    b_per_w = Bidx // nw
    n_chunks = b_per_w // chunk
    mesh = plsc.VectorSubcoreMesh(core_axis_name="c", subcore_axis_name="s")

    @functools.partial(
        pl.kernel, mesh=mesh,
        compiler_params=pltpu.CompilerParams(use_tc_tiling_on_sc=False),
        out_type=jax.ShapeDtypeStruct((Bidx, D), _F32),
        scratch_types=[
            pltpu.VMEM((chunk,), jnp.int32),
            pltpu.VMEM((chunk, D), _F32),
            pltpu.SemaphoreType.DMA,
        ],
    )
    def k(table_hbm, idx_hbm, out_hbm, idx_v, rows_v, sem):
        wid = lax.axis_index("s") * nc + lax.axis_index("c")
        base = wid * b_per_w
        for ch in range(n_chunks):
            off = base + ch * chunk
            pltpu.sync_copy(idx_hbm.at[pl.ds(off, chunk)], idx_v)
            pltpu.async_copy(table_hbm.at[idx_v], rows_v, sem).wait()
            pltpu.sync_copy(rows_v, out_hbm.at[pl.ds(off, chunk)])

    return k(table, idx_flat)


def _pc0_body(nbr_ref, q_ref, *wrefs, TQ, K):
    Ws = [wrefs[2 * i][...] for i in range(4)]
    bs = [wrefs[2 * i + 1][...] for i in range(4)]
    Fs = [wrefs[8 + 2 * i][...] for i in range(3)]
    fbs = [wrefs[9 + 2 * i][...] for i in range(3)]
    out_ref = wrefs[14]
    nbr = nbr_ref[0]
    q = q_ref[0]
    qr = jnp.broadcast_to(q[:, None, :], (TQ, K, 8)).reshape(TQ * K, 8)
    rel = nbr[:, 0:8] - qr
    w = _run_layers(rel, Ws, bs, norm=False, final_act=True)
    comb = jnp.sum(_b16(w).reshape(TQ, K, 128), axis=1) * (1.0 / K)
    f1 = _run_layers(comb, Fs, fbs, norm=True, final_act=False)
    out_ref[0] = f1


def _pc0(nbr, q_pad, wp, fp, TQ, K):
    B, M, _ = q_pad.shape
    grid = (B, M // TQ)
    args = []
    specs = [
        pl.BlockSpec((1, TQ * K, 16), lambda b, t: (b, t, 0)),
        pl.BlockSpec((1, TQ, 8), lambda b, t: (b, t, 0)),
    ]
    for l in wp + fp:
        for a in (l["W"], l["b"]):
            args.append(a)
            specs.append(pl.BlockSpec(a.shape, lambda b, t, _r=len(a.shape): (0,) * _r))
    return pl.pallas_call(
        functools.partial(_pc0_body, TQ=TQ, K=K),
        grid=grid,
        in_specs=specs,
        out_specs=pl.BlockSpec((1, TQ, 8), lambda b, t: (b, t, 0)),
        out_shape=jax.ShapeDtypeStruct((B, M, 8), _F32),
    )(nbr, q_pad, *args)


def _pc1_body(nbr_ref, q_ref, *wrefs, TQ, K):
    Ws = [wrefs[2 * i][...] for i in range(4)]
    bs = [wrefs[2 * i + 1][...] for i in range(4)]
    Fs = [wrefs[8 + 2 * i][...] for i in range(4)]
    fbs = [wrefs[9 + 2 * i][...] for i in range(4)]
    out_ref = wrefs[16]
    nbr = nbr_ref[0]
    q = q_ref[0]
    qr = jnp.broadcast_to(q[:, None, :], (TQ, K, 8)).reshape(TQ * K, 8)
    rel = nbr[:, 0:8] - qr
    w = _run_layers(rel, Ws, bs, norm=False, final_act=True)
    wb = _b16(w)
    parts = []
    for c in range(8):
        t = wb * _b16(nbr[:, 3 + c:4 + c])
        parts.append(jnp.sum(t.reshape(TQ, K, 64), axis=1))
    comb = jnp.concatenate(parts, axis=1) * (1.0 / K)
    f2 = _run_layers(comb, Fs, fbs, norm=True, final_act=False)
    out_ref[0] = f2


def _pc1(nbr, q1_pad, wp, fp, K):
    B, TQ, _ = q1_pad.shape
    args = []
    specs = [
        pl.BlockSpec((1, TQ * K, 16), lambda b: (b, 0, 0)),
        pl.BlockSpec((1, TQ, 8), lambda b: (b, 0, 0)),
    ]
    for l in wp + fp:
        for a in (l["W"], l["b"]):
            args.append(a)
            specs.append(pl.BlockSpec(a.shape, lambda b, _r=len(a.shape): (0,) * _r))
    return pl.pallas_call(
        functools.partial(_pc1_body, TQ=TQ, K=K),
        grid=(B,),
        in_specs=specs,
        out_specs=pl.BlockSpec((1, TQ, 32), lambda b: (b, 0, 0)),
        out_shape=jax.ShapeDtypeStruct((B, TQ, 32), _F32),
    )(nbr, q1_pad, *args)


def _pc2_body(q_ref, p1_ref, f2_ref, *wrefs, B):
    Ws = [wrefs[2 * i][...] for i in range(4)]
    bs = [wrefs[2 * i + 1][...] for i in range(4)]
    Fs = [wrefs[8 + 2 * i][...] for i in range(4)]
    fbs = [wrefs[9 + 2 * i][...] for i in range(4)]
    W1a = wrefs[16][...]
    b1 = wrefs[17][...]
    out_ref = wrefs[18]
    combs = []
    for b in range(B):
        qb = q_ref[b]
        mean = jnp.mean(qb, axis=0, keepdims=True)
        rel = p1_ref[b] - mean
        w2 = _run_layers(rel, Ws, bs, norm=False, final_act=True)
        f2b = _b16(f2_ref[b])
        w2t = jnp.concatenate([_b16(w2)] * 32, axis=1)
        f2r = jnp.concatenate(
            [jnp.broadcast_to(f2b[:, c:c + 1], (256, 64)) for c in range(32)],
            axis=1)
        combs.append(jnp.sum(f2r * w2t, axis=0, keepdims=True))
    comb = jnp.concatenate(combs, axis=0) * (1.0 / 256.0)
    feat = _run_layers(comb, Fs, fbs, norm=True, final_act=False)
    out_ref[...] = _mm(feat, W1a) + b1


def _pc2(q_pad, p1_pad, f2, wp, fp, W1a, b1):
    B = q_pad.shape[0]
    args = []
    specs = [
        pl.BlockSpec(q_pad.shape, lambda: (0, 0, 0)),
        pl.BlockSpec(p1_pad.shape, lambda: (0, 0, 0)),
        pl.BlockSpec(f2.shape, lambda: (0, 0, 0)),
    ]
    for l in wp + fp:
        for a in (l["W"], l["b"]):
            args.append(a)
            specs.append(pl.BlockSpec(a.shape, lambda _r=len(a.shape): (0,) * _r))
    args += [W1a, b1]
    specs += [pl.BlockSpec(W1a.shape, lambda: (0, 0)),
              pl.BlockSpec(b1.shape, lambda: (0, 0))]
    return pl.pallas_call(
        functools.partial(_pc2_body, B=B),
        grid=(),
        in_specs=specs,
        out_specs=pl.BlockSpec((B, 1024), lambda: (0, 0)),
        out_shape=jax.ShapeDtypeStruct((B, 1024), _F32),
    )(q_pad, p1_pad, f2, *args)


def _dec_body(noise_ref, base1_ref, *wrefs, TR):
    Ws = [wrefs[2 * i][...] for i in range(8)]
    bs = [wrefs[2 * i + 1][...] for i in range(8)]
    out_ref = wrefs[16]
    noise = noise_ref[0]
    h = _relu(_mm(noise, Ws[0]) + base1_ref[0])
    for i in range(1, 7):
        h = _relu(_mm(h, Ws[i]) + bs[i])
    out_ref[0] = _mm(h, Ws[7]) + bs[7]


def _decoder(noise_t, base1, W1b, dec, TR):
    B, P, _ = noise_t.shape
    base1_3d = base1.reshape(B, 1, 1024)
    args = [W1b, dec[0]["b"]]
    specs = [
        pl.BlockSpec((1, TR, 32), lambda b, t: (b, t, 0)),
        pl.BlockSpec((1, 1, 1024), lambda b, t: (b, 0, 0)),
        pl.BlockSpec(W1b.shape, lambda b, t: (0, 0)),
        pl.BlockSpec(dec[0]["b"].shape, lambda b, t: (0, 0)),
    ]
    for l in dec[1:]:
        for a in (l["W"], l["b"]):
            args.append(a)
            specs.append(pl.BlockSpec(a.shape, lambda b, t, _r=len(a.shape): (0,) * _r))
    return pl.pallas_call(
        functools.partial(_dec_body, TR=TR),
        grid=(B, P // TR),
        in_specs=specs,
        out_specs=pl.BlockSpec((1, TR, 3), lambda b, t: (b, t, 0)),
        out_shape=jax.ShapeDtypeStruct((B, P, 3), _F32),
    )(noise_t, base1_3d, *args)


def _prep(layers, pad_first=False):
    out = []
    for i, l in enumerate(layers):
        W = l["W"]
        if pad_first and i == 0:
            W = jnp.pad(W, ((0, 8 - W.shape[0]), (0, 0)))
        out.append({"W": W, "b": l["b"].reshape(1, -1)})
    return out


def kernel(points, params, out_count):
    B, N, _ = points.shape
    K = 32
    M1 = 256
    P = 1024

    q_pad = jnp.pad(points, ((0, 0), (0, 0), (0, 5)))
    pts_t = jnp.transpose(q_pad, (0, 2, 1))

    idx = _knn(pts_t, q_pad, K, 256)
    boff = (jnp.arange(B, dtype=jnp.int32) * N)[:, None, None]
    gidx0 = (idx + boff).reshape(-1)

    table0 = jnp.pad(points.reshape(B * N, 3), ((0, 0), (0, 13)))
    nbr0 = _sc_gather(table0, gidx0, 2048)

    wp0 = _prep(params["pc0_w"], pad_first=True)
    fp0 = _prep(params["pc0_f"])
    f1 = _pc0(nbr0.reshape(B, N * K, 16), q_pad, wp0, fp0, 256, K)

    table1 = jnp.concatenate(
        [points.reshape(B * N, 3), f1.reshape(B * N, 8),
         jnp.zeros((B * N, 5), _F32)], axis=1)
    gidx1 = (idx[:, :M1] + boff).reshape(-1)
    nbr1 = _sc_gather(table1, gidx1, 1024)

    wp1 = _prep(params["pc1_w"], pad_first=True)
    fp1 = _prep(params["pc1_f"])
    f2 = _pc1(nbr1.reshape(B, M1 * K, 16), q_pad[:, :M1], wp1, fp1, K)

    dec = params["dec"]
    W1a = dec[0]["W"][:256]
    W1b = dec[0]["W"][256:]
    wp2 = _prep(params["pc2_w"], pad_first=True)
    fp2 = _prep(params["pc2_f"])
    b1 = dec[0]["b"].reshape(1, -1)
    base1 = _pc2(q_pad, q_pad[:, :M1], f2, wp2, fp2, W1a, b1)

    noise = jax.random.normal(jax.random.key(1234), (B, 32, P), dtype=_F32)
    noise_t = jnp.transpose(noise, (0, 2, 1))
    dec_prep = _prep(dec)
    out = _decoder(noise_t, base1, W1b, dec_prep, 512)
    return out


# speedup vs baseline: 1.1729x; 1.1729x over previous
import functools

import jax
import jax.numpy as jnp
from jax import lax
from jax.experimental import pallas as pl
from jax.experimental.pallas import tpu as pltpu
from jax.experimental.pallas import tpu_sc as plsc

_F32 = jnp.float32
_HI = jax.lax.Precision.HIGHEST


def _bmm(a, b):
    return lax.dot_general(a.astype(jnp.bfloat16), b.astype(jnp.bfloat16),
                           (((1,), (0,)), ((), ())),
                           preferred_element_type=_F32)


def _mm(x, w, prec=None):
    if prec is not None:
        return lax.dot_general(x, w, (((1,), (0,)), ((), ())),
                               preferred_element_type=_F32, precision=prec)
    return _bmm(x, w)


def _b16(x):
    return x.astype(jnp.bfloat16).astype(_F32)


def _relu(x):
    return jnp.maximum(x, 0.0)


def _ln(x):
    m = jnp.mean(x, axis=-1, keepdims=True)
    v = jnp.mean((x - m) * (x - m), axis=-1, keepdims=True)
    return (x - m) / jnp.sqrt(v + 1e-5)


def _run_layers(x, Ws, bs, norm, final_act, prec=None):
    n = len(Ws)
    for i in range(n):
        x = _mm(x, Ws[i], prec=prec)
        x = x + bs[i]
        if i < n - 1 or final_act:
            if norm:
                x = _ln(x)
            x = _relu(x)
    return x


def _knn_body(pts_t_ref, q_ref, idx_ref, *, K):
    p = pts_t_ref[0]
    q = q_ref[0]
    pn = jnp.sum(p * p, axis=0, keepdims=True)
    qn = jnp.sum(q * q, axis=1, keepdims=True)
    qp = _mm(_b16(q), _b16(p), prec=_HI)
    d2 = jnp.maximum(qn + pn - 2.0 * qp, 0.0)
    iota = lax.broadcasted_iota(jnp.int32, d2.shape, 1)
    cols = []
    for _ in range(K):
        am = jnp.argmin(d2, axis=1).astype(jnp.int32)
        amc = am[:, None]
        cols.append(amc)
        d2 = jnp.where(iota == amc, jnp.inf, d2)
    idx_ref[0] = jnp.concatenate(cols, axis=1)


def _knn(pts_t, q_pad, K, TQ):
    B, _, N = pts_t.shape
    M = q_pad.shape[1]
    grid = (B, M // TQ)
    return pl.pallas_call(
        functools.partial(_knn_body, K=K),
        grid=grid,
        in_specs=[
            pl.BlockSpec((1, 8, N), lambda b, t: (b, 0, 0)),
            pl.BlockSpec((1, TQ, 8), lambda b, t: (b, t, 0)),
        ],
        out_specs=pl.BlockSpec((1, TQ, K), lambda b, t: (b, t, 0)),
        out_shape=jax.ShapeDtypeStruct((B, M, K), jnp.int32),
    )(pts_t, q_pad)


def _sc_gather(table, idx_flat, chunk):
    V, D = table.shape
    Bidx = idx_flat.shape[0]
    info = plsc.get_sparse_core_info()
    nc = info.num_cores
    nw = nc * info.num_subcores
    b_per_w = Bidx // nw
    n_chunks = b_per_w // chunk
    mesh = plsc.VectorSubcoreMesh(core_axis_name="c", subcore_axis_name="s")

    @functools.partial(
        pl.kernel, mesh=mesh,
        compiler_params=pltpu.CompilerParams(use_tc_tiling_on_sc=False),
        out_type=jax.ShapeDtypeStruct((Bidx, D), _F32),
        scratch_types=[
            pltpu.VMEM((chunk,), jnp.int32),
            pltpu.VMEM((chunk, D), _F32),
            pltpu.SemaphoreType.DMA,
        ],
    )
    def k(table_hbm, idx_hbm, out_hbm, idx_v, rows_v, sem):
        wid = lax.axis_index("s") * nc + lax.axis_index("c")
        base = wid * b_per_w
        for ch in range(n_chunks):
            off = base + ch * chunk
            pltpu.sync_copy(idx_hbm.at[pl.ds(off, chunk)], idx_v)
            pltpu.async_copy(table_hbm.at[idx_v], rows_v, sem).wait()
            pltpu.sync_copy(rows_v, out_hbm.at[pl.ds(off, chunk)])

    return k(table, idx_flat)


def _pc0_body(nbr_ref, q_ref, *wrefs, TQ, K):
    Ws = [wrefs[2 * i][...] for i in range(4)]
    bs = [wrefs[2 * i + 1][...] for i in range(4)]
    Fs = [wrefs[8 + 2 * i][...] for i in range(3)]
    fbs = [wrefs[9 + 2 * i][...] for i in range(3)]
    out_ref = wrefs[14]
    nbr = nbr_ref[0]
    q = q_ref[0]
    qr = jnp.broadcast_to(q[:, None, :], (TQ, K, 8)).reshape(TQ * K, 8)
    rel = nbr[:, 0:8] - qr
    w = _run_layers(rel, Ws, bs, norm=False, final_act=True)
    comb = jnp.sum(_b16(w).reshape(TQ, K, 128), axis=1) * (1.0 / K)
    f1 = _run_layers(comb, Fs, fbs, norm=True, final_act=False)
    out_ref[0] = f1


def _pc0(nbr, q_pad, wp, fp, TQ, K):
    B, M, _ = q_pad.shape
    grid = (B, M // TQ)
    args = []
    specs = [
        pl.BlockSpec((1, TQ * K, 16), lambda b, t: (b, t, 0)),
        pl.BlockSpec((1, TQ, 8), lambda b, t: (b, t, 0)),
    ]
    for l in wp + fp:
        for a in (l["W"], l["b"]):
            args.append(a)
            specs.append(pl.BlockSpec(a.shape, lambda b, t, _r=len(a.shape): (0,) * _r))
    return pl.pallas_call(
        functools.partial(_pc0_body, TQ=TQ, K=K),
        grid=grid,
        in_specs=specs,
        out_specs=pl.BlockSpec((1, TQ, 8), lambda b, t: (b, t, 0)),
        out_shape=jax.ShapeDtypeStruct((B, M, 8), _F32),
    )(nbr, q_pad, *args)


def _pc1_body(nbr_ref, q_ref, *wrefs, TQ, K):
    Ws = [wrefs[2 * i][...] for i in range(4)]
    bs = [wrefs[2 * i + 1][...] for i in range(4)]
    Fs = [wrefs[8 + 2 * i][...] for i in range(4)]
    fbs = [wrefs[9 + 2 * i][...] for i in range(4)]
    out_ref = wrefs[16]
    nbr = nbr_ref[0]
    q = q_ref[0]
    qr = jnp.broadcast_to(q[:, None, :], (TQ, K, 8)).reshape(TQ * K, 8)
    rel = nbr[:, 0:8] - qr
    w = _run_layers(rel, Ws, bs, norm=False, final_act=True)
    wb = _b16(w)
    parts = []
    for c in range(8):
        t = wb * _b16(nbr[:, 3 + c:4 + c])
        parts.append(jnp.sum(t.reshape(TQ, K, 64), axis=1))
    comb = jnp.concatenate(parts, axis=1) * (1.0 / K)
    f2 = _run_layers(comb, Fs, fbs, norm=True, final_act=False)
    out_ref[0] = f2


def _pc1(nbr, q1_pad, wp, fp, K):
    B, TQ, _ = q1_pad.shape
    args = []
    specs = [
        pl.BlockSpec((1, TQ * K, 16), lambda b: (b, 0, 0)),
        pl.BlockSpec((1, TQ, 8), lambda b: (b, 0, 0)),
    ]
    for l in wp + fp:
        for a in (l["W"], l["b"]):
            args.append(a)
            specs.append(pl.BlockSpec(a.shape, lambda b, _r=len(a.shape): (0,) * _r))
    return pl.pallas_call(
        functools.partial(_pc1_body, TQ=TQ, K=K),
        grid=(B,),
        in_specs=specs,
        out_specs=pl.BlockSpec((1, TQ, 32), lambda b: (b, 0, 0)),
        out_shape=jax.ShapeDtypeStruct((B, TQ, 32), _F32),
    )(nbr, q1_pad, *args)


def _pc2_body(q_ref, p1_ref, f2_ref, *wrefs, B):
    Ws = [wrefs[2 * i][...] for i in range(4)]
    bs = [wrefs[2 * i + 1][...] for i in range(4)]
    Fs = [wrefs[8 + 2 * i][...] for i in range(4)]
    fbs = [wrefs[9 + 2 * i][...] for i in range(4)]
    W1a = wrefs[16][...]
    b1 = wrefs[17][...]
    out_ref = wrefs[18]
    combs = []
    for b in range(B):
        qb = q_ref[b]
        mean = jnp.mean(qb, axis=0, keepdims=True)
        rel = p1_ref[b] - mean
        w2 = _run_layers(rel, Ws, bs, norm=False, final_act=True)
        f2b = _b16(f2_ref[b])
        w2t = jnp.concatenate([_b16(w2)] * 32, axis=1)
        f2r = jnp.concatenate(
            [jnp.broadcast_to(f2b[:, c:c + 1], (256, 64)) for c in range(32)],
            axis=1)
        combs.append(jnp.sum(f2r * w2t, axis=0, keepdims=True))
    comb = jnp.concatenate(combs, axis=0) * (1.0 / 256.0)
    feat = _run_layers(comb, Fs, fbs, norm=True, final_act=False)
    out_ref[...] = _mm(feat, W1a) + b1


def _pc2(q_pad, p1_pad, f2, wp, fp, W1a, b1):
    B = q_pad.shape[0]
    args = []
    specs = [
        pl.BlockSpec(q_pad.shape, lambda: (0, 0, 0)),
        pl.BlockSpec(p1_pad.shape, lambda: (0, 0, 0)),
        pl.BlockSpec(f2.shape, lambda: (0, 0, 0)),
    ]
    for l in wp + fp:
        for a in (l["W"], l["b"]):
            args.append(a)
            specs.append(pl.BlockSpec(a.shape, lambda _r=len(a.shape): (0,) * _r))
    args += [W1a, b1]
    specs += [pl.BlockSpec(W1a.shape, lambda: (0, 0)),
              pl.BlockSpec(b1.shape, lambda: (0, 0))]
    return pl.pallas_call(
        functools.partial(_pc2_body, B=B),
        grid=(),
        in_specs=specs,
        out_specs=pl.BlockSpec((B, 1024), lambda: (0, 0)),
        out_shape=jax.ShapeDtypeStruct((B, 1024), _F32),
    )(q_pad, p1_pad, f2, *args)


def _dec_body(noise_ref, base1_ref, *wrefs, TR):
    Ws = [wrefs[2 * i][...] for i in range(8)]
    bs = [wrefs[2 * i + 1][...] for i in range(8)]
    out_ref = wrefs[16]
    noise = noise_ref[0]
    h = _relu(_mm(noise, Ws[0]) + base1_ref[0])
    for i in range(1, 7):
        h = _relu(_mm(h, Ws[i]) + bs[i])
    out_ref[0] = _mm(h, Ws[7]) + bs[7]


def _decoder(noise_t, base1, W1b, dec, TR):
    B, P, _ = noise_t.shape
    base1_3d = base1.reshape(B, 1, 1024)
    args = [W1b, dec[0]["b"]]
    specs = [
        pl.BlockSpec((1, TR, 32), lambda b, t: (b, t, 0)),
        pl.BlockSpec((1, 1, 1024), lambda b, t: (b, 0, 0)),
        pl.BlockSpec(W1b.shape, lambda b, t: (0, 0)),
        pl.BlockSpec(dec[0]["b"].shape, lambda b, t: (0, 0)),
    ]
    for l in dec[1:]:
        for a in (l["W"], l["b"]):
            args.append(a)
            specs.append(pl.BlockSpec(a.shape, lambda b, t, _r=len(a.shape): (0,) * _r))
    return pl.pallas_call(
        functools.partial(_dec_body, TR=TR),
        grid=(B, P // TR),
        in_specs=specs,
        out_specs=pl.BlockSpec((1, TR, 3), lambda b, t: (b, t, 0)),
        out_shape=jax.ShapeDtypeStruct((B, P, 3), _F32),
    )(noise_t, base1_3d, *args)


def _prep(layers, pad_first=False):
    out = []
    for i, l in enumerate(layers):
        W = l["W"]
        if pad_first and i == 0:
            W = jnp.pad(W, ((0, 8 - W.shape[0]), (0, 0)))
        out.append({"W": W, "b": l["b"].reshape(1, -1)})
    return out


def kernel(points, params, out_count):
    B, N, _ = points.shape
    K = 32
    M1 = 256
    P = 1024

    q_pad = jnp.pad(points, ((0, 0), (0, 0), (0, 5)))
    pts_t = jnp.transpose(q_pad, (0, 2, 1))

    idx = _knn(pts_t, q_pad, K, 256)
    boff = (jnp.arange(B, dtype=jnp.int32) * N)[:, None, None]
    gidx0 = (idx + boff).reshape(-1)

    table0 = jnp.pad(points.reshape(B * N, 3), ((0, 0), (0, 13)))
    nbr0 = _sc_gather(table0, gidx0, 2048)

    wp0 = _prep(params["pc0_w"], pad_first=True)
    fp0 = _prep(params["pc0_f"])
    f1 = _pc0(nbr0.reshape(B, N * K, 16), q_pad, wp0, fp0, 256, K)

    table1 = jnp.concatenate(
        [points.reshape(B * N, 3), f1.reshape(B * N, 8),
         jnp.zeros((B * N, 5), _F32)], axis=1)
    gidx1 = (idx[:, :M1] + boff).reshape(-1)
    nbr1 = _sc_gather(table1, gidx1, 1024)

    wp1 = _prep(params["pc1_w"], pad_first=True)
    fp1 = _prep(params["pc1_f"])
    f2 = _pc1(nbr1.reshape(B, M1 * K, 16), q_pad[:, :M1], wp1, fp1, K)

    dec = params["dec"]
    W1a = dec[0]["W"][:256]
    W1b = dec[0]["W"][256:]
    wp2 = _prep(params["pc2_w"], pad_first=True)
    fp2 = _prep(params["pc2_f"])
    b1 = dec[0]["b"].reshape(1, -1)
    base1 = _pc2(q_pad, q_pad[:, :M1], f2, wp2, fp2, W1a, b1)

    noise = jax.random.normal(jax.random.key(1234), (B, 32, P), dtype=_F32)
    noise_t = jnp.transpose(noise, (0, 2, 1))
    dec_prep = _prep(dec)
    out = _decoder(noise_t, base1, W1b, dec_prep, 512)
    return out
